# Initial kernel scaffold; baseline (speedup 1.0000x reference)
import jax
import jax.numpy as jnp
from jax.experimental import pallas as pl


def kernel(x, W):
    raise NotImplementedError("write your pallas kernel here")


# speedup vs baseline: 1.6297x; 1.6297x over previous
import functools

import jax
import jax.numpy as jnp
from jax import lax
from jax.experimental import pallas as pl
from jax.experimental.pallas import tpu as pltpu
from jax.experimental.pallas import tpu_sc as plsc

N_TOK = 2048
N_E = 1024
E_DIM = 256
ALPHA = 0.9

BI = 128
NB = N_TOK // BI
NC = 32


def _tc_body(xt_ref, wt_ref, j_ref, loss_ref, acc_ref):
    b = pl.program_id(0)
    c = pl.program_id(1)

    xc = xt_ref[...]
    wc = wt_ref[...]
    xcT = xc.T

    t = []
    for k in range(8):
        d = xcT[:, k:k + 1] - wc[k:k + 1, :]
        t.append(d * d)
    g = ((t[0] + t[4]) + (t[2] + t[6])) + ((t[1] + t[5]) + (t[3] + t[7]))

    @pl.when(c == 0)
    def _():
        acc_ref[...] = g

    @pl.when(c > 0)
    def _():
        acc_ref[...] = acc_ref[...] + g

    @pl.when(c == NC - 1)
    def _():
        acc = acc_ref[...]
        m = jnp.min(acc, axis=1)
        iota = lax.broadcasted_iota(jnp.int32, (BI, N_E), 1)
        hit = jnp.where(acc == m[:, None], iota, jnp.int32(N_E))
        j_ref[...] = jnp.min(hit, axis=1)

        s = jnp.sum(m)

        @pl.when(b == 0)
        def _():
            loss_ref[0, 0] = 0.0

        loss_ref[0, 0] += s

        @pl.when(b == NB - 1)
        def _():
            loss_ref[0, 0] = loss_ref[0, 0] * ((1.0 + ALPHA) / N_TOK)


def _argmin_loss(xt, wt):
    return pl.pallas_call(
        _tc_body,
        grid=(NB, NC),
        in_specs=[
            pl.BlockSpec((8, BI), lambda b, c: (c, b)),
            pl.BlockSpec((8, N_E), lambda b, c: (c, 0)),
        ],
        out_specs=[
            pl.BlockSpec((BI,), lambda b, c: (b,)),
            pl.BlockSpec(memory_space=pltpu.SMEM, block_shape=(1, 1),
                         index_map=lambda b, c: (0, 0)),
        ],
        out_shape=[
            jax.ShapeDtypeStruct((N_TOK,), jnp.int32),
            jax.ShapeDtypeStruct((1, 1), jnp.float32),
        ],
        scratch_shapes=[pltpu.VMEM((BI, N_E), jnp.float32)],
    )(xt, wt)


def _sc_gather(W, j):
    info = plsc.get_sparse_core_info()
    ncores, nsub = info.num_cores, info.num_subcores
    nw = ncores * nsub
    bpw = N_TOK // nw
    mesh = plsc.VectorSubcoreMesh(core_axis_name="c", subcore_axis_name="s")

    @functools.partial(
        pl.kernel,
        mesh=mesh,
        out_type=jax.ShapeDtypeStruct((N_TOK, E_DIM), jnp.float32),
        scratch_types=[
            pltpu.VMEM((bpw,), jnp.int32),
            pltpu.VMEM((bpw, E_DIM), jnp.float32),
            pltpu.SemaphoreType.DMA,
        ],
    )
    def gather_k(w_hbm, idx_hbm, out_hbm, idx_v, rows_v, sem):
        wid = lax.axis_index("s") * ncores + lax.axis_index("c")
        base = wid * bpw
        pltpu.sync_copy(idx_hbm.at[pl.ds(base, bpw)], idx_v)
        pltpu.async_copy(w_hbm.at[idx_v], rows_v, sem).wait()
        pltpu.sync_copy(rows_v, out_hbm.at[pl.ds(base, bpw)])

    return gather_k(W, j)


def kernel(x, W):
    xt = x.T
    wt = W.T
    j, loss = _argmin_loss(xt, wt)
    W_j = _sc_gather(W, j)
    return (W_j, loss.reshape(()))


# speedup vs baseline: 2.1426x; 1.3147x over previous
import functools

import jax
import jax.numpy as jnp
from jax import lax
from jax.experimental import pallas as pl
from jax.experimental.pallas import tpu as pltpu
from jax.experimental.pallas import tpu_sc as plsc

N_TOK = 2048
N_E = 1024
E_DIM = 256
ALPHA = 0.9

BI = 256
NB = N_TOK // BI
NC = E_DIM // 8


def _tc_body(xt_ref, wt_ref, j_ref, loss_ref, acc_ref):
    b = pl.program_id(0)
    c = pl.program_id(1)

    xc = xt_ref[...]
    wc = wt_ref[...]
    xcT = xc.T

    @pl.when(c == 0)
    def _():
        acc_ref[...] = jnp.zeros((BI, N_E), jnp.float32)

    for s in range(BI // 8):
        xs = xcT[8 * s:8 * s + 8, :]

        def sq(k):
            d = xs[:, k:k + 1] - wc[k:k + 1, :]
            return d * d

        g = ((sq(0) + sq(4)) + (sq(2) + sq(6))) + \
            ((sq(1) + sq(5)) + (sq(3) + sq(7)))
        row = pl.ds(8 * s, 8)
        acc_ref[row, :] = acc_ref[row, :] + g

    @pl.when(c == NC - 1)
    def _():
        acc = acc_ref[...]
        m = jnp.min(acc, axis=1)
        iota = lax.broadcasted_iota(jnp.int32, (BI, N_E), 1)
        hit = jnp.where(acc == m[:, None], iota, jnp.int32(N_E))
        j_ref[...] = jnp.min(hit, axis=1)

        s = jnp.sum(m)

        @pl.when(b == 0)
        def _():
            loss_ref[0, 0] = 0.0

        loss_ref[0, 0] += s

        @pl.when(b == NB - 1)
        def _():
            loss_ref[0, 0] = loss_ref[0, 0] * ((1.0 + ALPHA) / N_TOK)


def _argmin_loss(xt, wt):
    return pl.pallas_call(
        _tc_body,
        grid=(NB, NC),
        in_specs=[
            pl.BlockSpec((8, BI), lambda b, c: (c, b)),
            pl.BlockSpec((8, N_E), lambda b, c: (c, 0)),
        ],
        out_specs=[
            pl.BlockSpec((BI,), lambda b, c: (b,)),
            pl.BlockSpec(memory_space=pltpu.SMEM, block_shape=(1, 1),
                         index_map=lambda b, c: (0, 0)),
        ],
        out_shape=[
            jax.ShapeDtypeStruct((N_TOK,), jnp.int32),
            jax.ShapeDtypeStruct((1, 1), jnp.float32),
        ],
        scratch_shapes=[pltpu.VMEM((BI, N_E), jnp.float32)],
    )(xt, wt)


def _sc_gather(W, j):
    info = plsc.get_sparse_core_info()
    ncores, nsub = info.num_cores, info.num_subcores
    nw = ncores * nsub
    bpw = N_TOK // nw
    mesh = plsc.VectorSubcoreMesh(core_axis_name="c", subcore_axis_name="s")

    @functools.partial(
        pl.kernel,
        mesh=mesh,
        out_type=jax.ShapeDtypeStruct((N_TOK, E_DIM), jnp.float32),
        scratch_types=[
            pltpu.VMEM((bpw,), jnp.int32),
            pltpu.VMEM((bpw, E_DIM), jnp.float32),
            pltpu.SemaphoreType.DMA,
        ],
    )
    def gather_k(w_hbm, idx_hbm, out_hbm, idx_v, rows_v, sem):
        wid = lax.axis_index("s") * ncores + lax.axis_index("c")
        base = wid * bpw
        pltpu.sync_copy(idx_hbm.at[pl.ds(base, bpw)], idx_v)
        pltpu.async_copy(w_hbm.at[idx_v], rows_v, sem).wait()
        pltpu.sync_copy(rows_v, out_hbm.at[pl.ds(base, bpw)])

    return gather_k(W, j)


def kernel(x, W):
    xt = x.T
    wt = W.T
    j, loss = _argmin_loss(xt, wt)
    W_j = _sc_gather(W, j)
    return (W_j, loss.reshape(()))


# speedup vs baseline: 2.3741x; 1.1081x over previous
import functools

import jax
import jax.numpy as jnp
from jax import lax
from jax.experimental import pallas as pl
from jax.experimental.pallas import tpu as pltpu
from jax.experimental.pallas import tpu_sc as plsc

N_TOK = 2048
N_E = 1024
E_DIM = 256
ALPHA = 0.9

BI = 512
NB = N_TOK // BI
NC = E_DIM // 8


def _tc_body(xt_ref, wt_ref, j_ref, loss_ref, acc_ref):
    b = pl.program_id(0)
    c = pl.program_id(1)

    xc = xt_ref[...]
    wc = wt_ref[...]
    xcT = xc.T

    @pl.when(c == 0)
    def _():
        acc_ref[...] = jnp.zeros((BI, N_E), jnp.float32)

    for s in range(BI // 8):
        xs = xcT[8 * s:8 * s + 8, :]

        def sq(k):
            d = xs[:, k:k + 1] - wc[k:k + 1, :]
            return d * d

        g = ((sq(0) + sq(4)) + (sq(2) + sq(6))) + \
            ((sq(1) + sq(5)) + (sq(3) + sq(7)))
        row = pl.ds(8 * s, 8)
        acc_ref[row, :] = acc_ref[row, :] + g

    @pl.when(c == NC - 1)
    def _():
        acc = acc_ref[...]
        m = jnp.min(acc, axis=1)
        iota = lax.broadcasted_iota(jnp.int32, (BI, N_E), 1)
        hit = jnp.where(acc == m[:, None], iota, jnp.int32(N_E))
        j_ref[...] = jnp.min(hit, axis=1)

        s = jnp.sum(m)

        @pl.when(b == 0)
        def _():
            loss_ref[0, 0] = 0.0

        loss_ref[0, 0] += s

        @pl.when(b == NB - 1)
        def _():
            loss_ref[0, 0] = loss_ref[0, 0] * ((1.0 + ALPHA) / N_TOK)


def _argmin_loss(xt, wt):
    return pl.pallas_call(
        _tc_body,
        grid=(NB, NC),
        in_specs=[
            pl.BlockSpec((8, BI), lambda b, c: (c, b)),
            pl.BlockSpec((8, N_E), lambda b, c: (c, 0)),
        ],
        out_specs=[
            pl.BlockSpec((BI,), lambda b, c: (b,)),
            pl.BlockSpec(memory_space=pltpu.SMEM, block_shape=(1, 1),
                         index_map=lambda b, c: (0, 0)),
        ],
        out_shape=[
            jax.ShapeDtypeStruct((N_TOK,), jnp.int32),
            jax.ShapeDtypeStruct((1, 1), jnp.float32),
        ],
        scratch_shapes=[pltpu.VMEM((BI, N_E), jnp.float32)],
    )(xt, wt)


def _sc_gather(W, j):
    info = plsc.get_sparse_core_info()
    ncores, nsub = info.num_cores, info.num_subcores
    nw = ncores * nsub
    bpw = N_TOK // nw
    mesh = plsc.VectorSubcoreMesh(core_axis_name="c", subcore_axis_name="s")

    @functools.partial(
        pl.kernel,
        mesh=mesh,
        out_type=jax.ShapeDtypeStruct((N_TOK, E_DIM), jnp.float32),
        scratch_types=[
            pltpu.VMEM((bpw,), jnp.int32),
            pltpu.VMEM((bpw, E_DIM), jnp.float32),
            pltpu.SemaphoreType.DMA,
        ],
    )
    def gather_k(w_hbm, idx_hbm, out_hbm, idx_v, rows_v, sem):
        wid = lax.axis_index("s") * ncores + lax.axis_index("c")
        base = wid * bpw
        pltpu.sync_copy(idx_hbm.at[pl.ds(base, bpw)], idx_v)
        pltpu.async_copy(w_hbm.at[idx_v], rows_v, sem).wait()
        pltpu.sync_copy(rows_v, out_hbm.at[pl.ds(base, bpw)])

    return gather_k(W, j)


def kernel(x, W):
    xt = x.T
    wt = W.T
    j, loss = _argmin_loss(xt, wt)
    W_j = _sc_gather(W, j)
    return (W_j, loss.reshape(()))


# speedup vs baseline: 2.9638x; 1.2484x over previous
import functools

import jax
import jax.numpy as jnp
from jax import lax
from jax.experimental import pallas as pl
from jax.experimental.pallas import tpu as pltpu
from jax.experimental.pallas import tpu_sc as plsc

N_TOK = 2048
N_E = 1024
E_DIM = 256
ALPHA = 0.9

R_TC = 1536
NS_ROWS = N_TOK - R_TC
BI = 512
NB = R_TC // BI
NC = E_DIM // 8

RPT = 16
NPASS = 4
CPP = N_E // NPASS
NJG = CPP // 16


def _tc_body(xt_ref, wt_ref, j_ref, part_ref, acc_ref):
    b = pl.program_id(0)
    c = pl.program_id(1)

    xc = xt_ref[...]
    wc = wt_ref[...]
    xcT = xc.T

    @pl.when(c == 0)
    def _():
        acc_ref[...] = jnp.zeros((BI, N_E), jnp.float32)

    for s in range(BI // 8):
        xs = xcT[8 * s:8 * s + 8, :]

        def sq(k):
            d = xs[:, k:k + 1] - wc[k:k + 1, :]
            return d * d

        g = ((sq(0) + sq(4)) + (sq(2) + sq(6))) + \
            ((sq(1) + sq(5)) + (sq(3) + sq(7)))
        row = pl.ds(8 * s, 8)
        acc_ref[row, :] = acc_ref[row, :] + g

    @pl.when(c == NC - 1)
    def _():
        acc = acc_ref[...]
        m = jnp.min(acc, axis=1)
        iota = lax.broadcasted_iota(jnp.int32, (BI, N_E), 1)
        hit = jnp.where(acc == m[:, None], iota, jnp.int32(N_E))
        j_ref[...] = jnp.min(hit, axis=1)

        s = jnp.sum(m)

        @pl.when(b == 0)
        def _():
            part_ref[0, 0] = 0.0

        part_ref[0, 0] += s


def _argmin_tc(xt, wt):
    return pl.pallas_call(
        _tc_body,
        grid=(NB, NC),
        in_specs=[
            pl.BlockSpec((8, BI), lambda b, c: (c, b)),
            pl.BlockSpec((8, N_E), lambda b, c: (c, 0)),
        ],
        out_specs=[
            pl.BlockSpec((BI,), lambda b, c: (b,)),
            pl.BlockSpec(memory_space=pltpu.SMEM, block_shape=(1, 1),
                         index_map=lambda b, c: (0, 0)),
        ],
        out_shape=[
            jax.ShapeDtypeStruct((R_TC,), jnp.int32),
            jax.ShapeDtypeStruct((1, 1), jnp.float32),
        ],
        scratch_shapes=[pltpu.VMEM((BI, N_E), jnp.float32)],
    )(xt, wt)


def _argmin_sc(x, w4):
    info = plsc.get_sparse_core_info()
    ncores = info.num_cores
    mesh = plsc.VectorSubcoreMesh(core_axis_name="c", subcore_axis_name="s")

    @functools.partial(
        pl.kernel,
        mesh=mesh,
        out_type=[
            jax.ShapeDtypeStruct((NS_ROWS, 16), jnp.float32),
            jax.ShapeDtypeStruct((NS_ROWS, 16), jnp.int32),
        ],
        scratch_types=[
            pltpu.VMEM((RPT, E_DIM), jnp.float32),
            pltpu.VMEM((E_DIM, CPP), jnp.float32),
            pltpu.VMEM((RPT, 16), jnp.float32),
            pltpu.VMEM((RPT, 16), jnp.int32),
            pltpu.SemaphoreType.DMA,
            pltpu.SemaphoreType.DMA,
        ],
    )
    def dist_k(x_hbm, w4_hbm, rm_hbm, ri_hbm,
               x_vm, wt_vm, rm_vm, ri_vm, semx, semw):
        wid = lax.axis_index("s") * ncores + lax.axis_index("c")
        base = wid * RPT
        pltpu.async_copy(x_hbm.at[pl.ds(R_TC + base, RPT)], x_vm, semx).wait()

        iota = lax.iota(jnp.int32, 16)

        def init_row(r, _):
            rm_vm[r, :] = jnp.full((16,), jnp.float32(jnp.inf))
            ri_vm[r, :] = jnp.full((16,), jnp.int32(N_E))
            return 0

        lax.fori_loop(0, RPT, init_row, 0)

        def do_pass(p, _):
            pltpu.async_copy(w4_hbm.at[p], wt_vm, semw).wait()

            def do_row(r, _):
                accs = tuple(jnp.zeros((16,), jnp.float32)
                             for _ in range(NJG))

                def chunk2(cc, accs):
                    accs = list(accs)
                    xv = x_vm[r, pl.ds(16 * cc, 16)]
                    for half in range(2):
                        xsc = [jnp.broadcast_to(xv[8 * half + k], (16,))
                               for k in range(8)]
                        for jg in range(NJG):
                            def sq(k):
                                e = 16 * cc + 8 * half + k
                                d = wt_vm[e, pl.ds(16 * jg, 16)] - xsc[k]
                                return d * d

                            g = ((sq(0) + sq(4)) + (sq(2) + sq(6))) + \
                                ((sq(1) + sq(5)) + (sq(3) + sq(7)))
                            accs[jg] = accs[jg] + g
                    return tuple(accs)

                accs = lax.fori_loop(0, NC // 2, chunk2, accs)
                rm = rm_vm[r, :]
                ri = ri_vm[r, :]
                for jg in range(NJG):
                    gidx = iota + (p * CPP + jg * 16)
                    lt = accs[jg] < rm
                    rm = jnp.where(lt, accs[jg], rm)
                    ri = jnp.where(lt, gidx, ri)
                rm_vm[r, :] = rm
                ri_vm[r, :] = ri
                return 0

            lax.fori_loop(0, RPT, do_row, 0)
            return 0

        lax.fori_loop(0, NPASS, do_pass, 0)

        pltpu.sync_copy(rm_vm, rm_hbm.at[pl.ds(base, RPT)])
        pltpu.sync_copy(ri_vm, ri_hbm.at[pl.ds(base, RPT)])

    return dist_k(x, w4)


def _finish_body(part_ref, rm_ref, ri_ref, j_ref, loss_ref):
    rm = rm_ref[...]
    mv = jnp.min(rm, axis=1)
    cand = jnp.where(rm == mv[:, None], ri_ref[...], jnp.int32(N_E))
    j_ref[...] = jnp.min(cand, axis=1)
    loss_ref[0, 0] = (part_ref[0, 0] + jnp.sum(mv)) * ((1.0 + ALPHA) / N_TOK)


def _finish(part, rm, ri):
    return pl.pallas_call(
        _finish_body,
        in_specs=[
            pl.BlockSpec(memory_space=pltpu.SMEM),
            pl.BlockSpec(memory_space=pltpu.VMEM),
            pl.BlockSpec(memory_space=pltpu.VMEM),
        ],
        out_specs=[
            pl.BlockSpec(memory_space=pltpu.VMEM),
            pl.BlockSpec(memory_space=pltpu.SMEM),
        ],
        out_shape=[
            jax.ShapeDtypeStruct((NS_ROWS,), jnp.int32),
            jax.ShapeDtypeStruct((1, 1), jnp.float32),
        ],
    )(part, rm, ri)


def _sc_gather(W, j):
    info = plsc.get_sparse_core_info()
    ncores, nsub = info.num_cores, info.num_subcores
    nw = ncores * nsub
    bpw = N_TOK // nw
    mesh = plsc.VectorSubcoreMesh(core_axis_name="c", subcore_axis_name="s")

    @functools.partial(
        pl.kernel,
        mesh=mesh,
        out_type=jax.ShapeDtypeStruct((N_TOK, E_DIM), jnp.float32),
        scratch_types=[
            pltpu.VMEM((bpw,), jnp.int32),
            pltpu.VMEM((bpw, E_DIM), jnp.float32),
            pltpu.SemaphoreType.DMA,
        ],
    )
    def gather_k(w_hbm, idx_hbm, out_hbm, idx_v, rows_v, sem):
        wid = lax.axis_index("s") * ncores + lax.axis_index("c")
        base = wid * bpw
        pltpu.sync_copy(idx_hbm.at[pl.ds(base, bpw)], idx_v)
        pltpu.async_copy(w_hbm.at[idx_v], rows_v, sem).wait()
        pltpu.sync_copy(rows_v, out_hbm.at[pl.ds(base, bpw)])

    return gather_k(W, j)


def kernel(x, W):
    xt = x.T
    wt = W.T
    w4 = wt.reshape(E_DIM, NPASS, CPP).transpose(1, 0, 2)
    rm, ri = _argmin_sc(x, w4)
    j_tc, part = _argmin_tc(xt, wt)
    j_sc, loss = _finish(part, rm, ri)
    j = jnp.concatenate([j_tc, j_sc])
    W_j = _sc_gather(W, j)
    return (W_j, loss.reshape(()))
